# baseline (speedup 1.0000x reference)
import functools

import jax
import jax.numpy as jnp
from jax import lax
from jax.experimental import pallas as pl
from jax.experimental.pallas import tpu as pltpu
from jax.experimental.pallas import tpu_sc as plsc

N = 10000
E = 320000
IN_DIM = 128
OUT_DIM = 16

NC = 2
NS = 16
NW = NC * NS
NPAD = 10240
ET = E // NW
ETP = 10240
ROWS = ETP // 128
NDEPTH = 8
NODES_PER_TILE = NPAD // NS
FIN_PER_TILE = NPAD // NW
BS = 2048


def _zsd_body(th_ref, h_ref, w_ref, a_ref, z_ref, s_ref, d_ref):
    th = th_ref[0, 0]
    w = w_ref[...]
    wm = w * (jnp.abs(w) > th).astype(w.dtype)
    z = lax.dot_general(h_ref[...], wm, (((1,), (1,)), ((), ())),
                        preferred_element_type=jnp.float32)
    a = a_ref[...]
    am = a * (jnp.abs(a) > th).astype(a.dtype)
    a1 = am[:, :OUT_DIM]
    a2 = am[:, OUT_DIM:]
    z_ref[...] = z
    s_ref[...] = jnp.sum(z * a1, axis=1)
    d_ref[...] = jnp.sum(z * a2, axis=1)


_zsd_call = pl.pallas_call(
    _zsd_body,
    grid=(NPAD // BS,),
    in_specs=[
        pl.BlockSpec((1, 1), lambda i: (0, 0)),
        pl.BlockSpec((BS, IN_DIM), lambda i: (i, 0)),
        pl.BlockSpec((OUT_DIM, IN_DIM), lambda i: (0, 0)),
        pl.BlockSpec((1, 2 * OUT_DIM), lambda i: (0, 0)),
    ],
    out_specs=[
        pl.BlockSpec((BS, OUT_DIM), lambda i: (i, 0)),
        pl.BlockSpec((BS,), lambda i: (i,)),
        pl.BlockSpec((BS,), lambda i: (i,)),
    ],
    out_shape=[
        jax.ShapeDtypeStruct((NPAD, OUT_DIM), jnp.float32),
        jax.ShapeDtypeStruct((NPAD,), jnp.float32),
        jax.ShapeDtypeStruct((NPAD,), jnp.float32),
    ],
)


def _edge_body(edge_hbm, z_hbm, s_hbm, d_hbm, accp, denp,
               s_loc, d_loc, src1, dst1, dst2d, den_loc,
               zrows, wrows, zbuf, acc_sh, gsems, ssems):
    cid = lax.axis_index("c")
    sid = lax.axis_index("s")
    wid = cid * NS + sid
    ebase = wid * ET
    zero16 = jnp.zeros((16,), jnp.float32)
    zero16i = jnp.zeros((16,), jnp.int32)

    padn = jnp.full((16,), N, jnp.int32)
    def ztail(i, carry):
        src1[pl.ds(ET + i * 16, 16)] = zero16i
        dst1[pl.ds(ET + i * 16, 16)] = padn
        s_loc[pl.ds(N + i * 16, 16)] = zero16
        d_loc[pl.ds(N + i * 16, 16)] = zero16
        return carry
    lax.fori_loop(0, (ETP - ET) // 16, ztail, 0)

    cp_s = pltpu.async_copy(s_hbm.at[pl.ds(0, N)], s_loc.at[pl.ds(0, N)], gsems[0])
    cp_d = pltpu.async_copy(d_hbm.at[pl.ds(0, N)], d_loc.at[pl.ds(0, N)], gsems[1])
    cp_src = pltpu.async_copy(edge_hbm.at[0, pl.ds(ebase, ET)],
                              src1.at[pl.ds(0, ET)], gsems[2])
    cp_dst = pltpu.async_copy(edge_hbm.at[1, pl.ds(ebase, ET)],
                              dst1.at[pl.ds(0, ET)], gsems[3])

    def zloop(i, carry):
        zbuf[i, :] = zero16
        den_loc[pl.ds(i * 16, 16)] = zero16
        return carry
    lax.fori_loop(0, NPAD // 16, zloop, 0)

    pltpu.sync_copy(zbuf, acc_sh.at[pl.ds(sid * NODES_PER_TILE, NODES_PER_TILE)])
    cp_dst.wait()

    def rloop(i, carry):
        v = dst1[pl.ds(i * 16, 16)]
        dst2d[i // 8, pl.ds((i % 8) * 16, 16)] = v
        return carry
    lax.fori_loop(0, ETP // 16, rloop, 0)
    cp_s.wait()
    cp_d.wait()
    cp_src.wait()
    plsc.subcore_barrier()

    bufs = [(zrows[b], wrows[b], gsems[b], ssems[b]) for b in range(NDEPTH)]
    for b, (zb, wb, gs, ss) in enumerate(bufs):
        pltpu.async_copy(z_hbm.at[src1.at[pl.ds(b * 128, 128)]], zb, gs)

    def body2(jj, carry):
        for b, (zb, wb, gs, ss) in enumerate(bufs):
            j = jj * NDEPTH + b
            pltpu.make_async_copy(z_hbm.at[src1.at[pl.ds(0, 128)]], zb, gs).wait()

            @pl.when(jj > 0)
            def _():
                pltpu.make_async_copy(wb, acc_sh.at[dst2d.at[j]], ss).wait()

            @plsc.parallel_loop(0, 8, 1, unroll=2)
            def chunk(k):
                off = j * 128 + k * 16
                si = src1[pl.ds(off, 16)]
                di = dst1[pl.ds(off, 16)]
                sv = plsc.load_gather(s_loc, [si])
                dv = plsc.load_gather(d_loc, [di])
                e = sv + dv
                e = jnp.where(e > 0, e, e * jnp.float32(0.01))
                ex = jnp.exp(e)
                plsc.addupdate_scatter(den_loc, [di], ex)
                base = k * 16
                for t in range(16):
                    wb[base + t, :] = zb[base + t, :] * ex[t]
            pltpu.async_copy(wb, acc_sh.at[dst2d.at[j]], ss, add=True)

            @pl.when(j + NDEPTH < ROWS)
            def _():
                pltpu.async_copy(
                    z_hbm.at[src1.at[pl.ds((j + NDEPTH) * 128, 128)]], zb, gs)
        return carry
    lax.fori_loop(0, ROWS // NDEPTH, body2, 0)
    pltpu.sync_copy(den_loc, denp.at[cid, sid])
    for b, (zb, wb, gs, ss) in enumerate(bufs):
        pltpu.make_async_copy(wb, acc_sh.at[dst2d.at[b]], ss).wait()
    plsc.subcore_barrier()

    nbase = sid * NODES_PER_TILE
    pltpu.sync_copy(acc_sh.at[pl.ds(nbase, NODES_PER_TILE)],
                    accp.at[cid, pl.ds(nbase, NODES_PER_TILE)])


_edge_call = functools.partial(
    pl.kernel,
    out_type=(jax.ShapeDtypeStruct((NC, NPAD, OUT_DIM), jnp.float32),
              jax.ShapeDtypeStruct((NC, NS, NPAD), jnp.float32)),
    mesh=plsc.VectorSubcoreMesh(core_axis_name="c", subcore_axis_name="s",
                                num_cores=NC, num_subcores=NS),
    scratch_types=[
        pltpu.VMEM((NPAD,), jnp.float32),
        pltpu.VMEM((NPAD,), jnp.float32),
        pltpu.VMEM((ETP,), jnp.int32),
        pltpu.VMEM((ETP,), jnp.int32),
        pltpu.VMEM((ROWS, 128), jnp.int32),
        pltpu.VMEM((NPAD,), jnp.float32),
        [pltpu.VMEM((128, OUT_DIM), jnp.float32) for _ in range(NDEPTH)],
        [pltpu.VMEM((128, OUT_DIM), jnp.float32) for _ in range(NDEPTH)],
        pltpu.VMEM((NODES_PER_TILE, OUT_DIM), jnp.float32),
        pltpu.VMEM_SHARED((NPAD, OUT_DIM), jnp.float32),
        [pltpu.SemaphoreType.DMA for _ in range(NDEPTH)],
        [pltpu.SemaphoreType.DMA for _ in range(NDEPTH)],
    ],
    compiler_params=pltpu.CompilerParams(needs_layout_passes=False,
                                         use_tc_tiling_on_sc=False),
)(_edge_body)


def _fin_body(accp, denp, out, a0b, a1b, denb, outb, sem0, sem1, sem2):
    cid = lax.axis_index("c")
    sid = lax.axis_index("s")
    wid = cid * NS + sid
    nbase = wid * FIN_PER_TILE

    cp0 = pltpu.async_copy(accp.at[0, pl.ds(nbase, FIN_PER_TILE)], a0b, sem0)
    cp1 = pltpu.async_copy(accp.at[1, pl.ds(nbase, FIN_PER_TILE)], a1b, sem1)
    cp2 = pltpu.async_copy(denp.at[:, :, pl.ds(nbase, FIN_PER_TILE)], denb, sem2)
    cp0.wait()
    cp1.wait()
    cp2.wait()

    eps = jnp.float32(1e-16)
    one = jnp.float32(1.0)

    def body(v, carry):
        dtot = denb[0, 0, pl.ds(v * 16, 16)]
        for c in range(NC):
            for t in range(NS):
                if c == 0 and t == 0:
                    continue
                dtot = dtot + denb[c, t, pl.ds(v * 16, 16)]
        dtot = dtot + eps
        base = v * 16
        for t in range(16):
            y = (a0b[base + t, :] + a1b[base + t, :]) / dtot[t]
            y = jnp.where(y > 0, y, jnp.exp(y) - one)
            outb[base + t, :] = y
        return carry
    lax.fori_loop(0, FIN_PER_TILE // 16, body, 0)

    @pl.when(nbase + FIN_PER_TILE <= N)
    def _():
        pltpu.sync_copy(outb, out.at[pl.ds(nbase, FIN_PER_TILE)])

    @pl.when(nbase + FIN_PER_TILE > N)
    def _():
        pltpu.sync_copy(outb.at[pl.ds(0, N - (NW - 1) * FIN_PER_TILE)],
                        out.at[pl.ds((NW - 1) * FIN_PER_TILE,
                                     N - (NW - 1) * FIN_PER_TILE)])


_fin_call = functools.partial(
    pl.kernel,
    out_type=jax.ShapeDtypeStruct((N, OUT_DIM), jnp.float32),
    mesh=plsc.VectorSubcoreMesh(core_axis_name="c", subcore_axis_name="s",
                                num_cores=NC, num_subcores=NS),
    scratch_types=[
        pltpu.VMEM((FIN_PER_TILE, OUT_DIM), jnp.float32),
        pltpu.VMEM((FIN_PER_TILE, OUT_DIM), jnp.float32),
        pltpu.VMEM((NC, NS, FIN_PER_TILE), jnp.float32),
        pltpu.VMEM((FIN_PER_TILE, OUT_DIM), jnp.float32),
        pltpu.SemaphoreType.DMA,
        pltpu.SemaphoreType.DMA,
        pltpu.SemaphoreType.DMA,
    ],
    compiler_params=pltpu.CompilerParams(needs_layout_passes=False,
                                         use_tc_tiling_on_sc=False),
)(_fin_body)


def kernel(h, edge_index, threshold, W, A):
    th2 = jnp.reshape(threshold.astype(jnp.float32), (1, 1))
    z, s, d = _zsd_call(th2, h, W, A)
    accp, denp = _edge_call(edge_index, z, s, d)
    return _fin_call(accp, denp)


# speedup vs baseline: 1.0316x; 1.0316x over previous
import functools

import jax
import jax.numpy as jnp
from jax import lax
from jax.experimental import pallas as pl
from jax.experimental.pallas import tpu as pltpu
from jax.experimental.pallas import tpu_sc as plsc

N = 10000
E = 320000
IN_DIM = 128
OUT_DIM = 16

NC = 2
NS = 16
NW = NC * NS
NPAD = 10240
ET = E // NW
ETP = 10240
ROWS = ETP // 128
NDEPTH = 4
NODES_PER_TILE = NPAD // NS
FIN_PER_TILE = NPAD // NW
BS = 2048


def _zsd_body(th_ref, h_ref, w_ref, a_ref, z_ref, s_ref, d_ref):
    th = th_ref[0, 0]
    w = w_ref[...]
    wm = w * (jnp.abs(w) > th).astype(w.dtype)
    z = lax.dot_general(h_ref[...], wm, (((1,), (1,)), ((), ())),
                        preferred_element_type=jnp.float32)
    a = a_ref[...]
    am = a * (jnp.abs(a) > th).astype(a.dtype)
    a1 = am[:, :OUT_DIM]
    a2 = am[:, OUT_DIM:]
    z_ref[...] = z
    s_ref[...] = jnp.sum(z * a1, axis=1)
    d_ref[...] = jnp.sum(z * a2, axis=1)


_zsd_call = pl.pallas_call(
    _zsd_body,
    grid=(NPAD // BS,),
    in_specs=[
        pl.BlockSpec((1, 1), lambda i: (0, 0)),
        pl.BlockSpec((BS, IN_DIM), lambda i: (i, 0)),
        pl.BlockSpec((OUT_DIM, IN_DIM), lambda i: (0, 0)),
        pl.BlockSpec((1, 2 * OUT_DIM), lambda i: (0, 0)),
    ],
    out_specs=[
        pl.BlockSpec((BS, OUT_DIM), lambda i: (i, 0)),
        pl.BlockSpec((BS,), lambda i: (i,)),
        pl.BlockSpec((BS,), lambda i: (i,)),
    ],
    out_shape=[
        jax.ShapeDtypeStruct((NPAD, OUT_DIM), jnp.float32),
        jax.ShapeDtypeStruct((NPAD,), jnp.float32),
        jax.ShapeDtypeStruct((NPAD,), jnp.float32),
    ],
)


def _edge_body(edge_hbm, z_hbm, s_hbm, d_hbm, accp, denp,
               s_loc, d_loc, src1, dst1, dst2d, den_loc,
               zrows, wrows, zbuf, acc_sh, gsems, ssems):
    cid = lax.axis_index("c")
    sid = lax.axis_index("s")
    wid = cid * NS + sid
    ebase = wid * ET
    zero16 = jnp.zeros((16,), jnp.float32)
    zero16i = jnp.zeros((16,), jnp.int32)

    padn = jnp.full((16,), N, jnp.int32)
    def ztail(i, carry):
        src1[pl.ds(ET + i * 16, 16)] = zero16i
        dst1[pl.ds(ET + i * 16, 16)] = padn
        s_loc[pl.ds(N + i * 16, 16)] = zero16
        d_loc[pl.ds(N + i * 16, 16)] = zero16
        return carry
    lax.fori_loop(0, (ETP - ET) // 16, ztail, 0)

    cp_s = pltpu.async_copy(s_hbm.at[pl.ds(0, N)], s_loc.at[pl.ds(0, N)], gsems[0])
    cp_d = pltpu.async_copy(d_hbm.at[pl.ds(0, N)], d_loc.at[pl.ds(0, N)], gsems[1])
    cp_src = pltpu.async_copy(edge_hbm.at[0, pl.ds(ebase, ET)],
                              src1.at[pl.ds(0, ET)], gsems[2])
    cp_dst = pltpu.async_copy(edge_hbm.at[1, pl.ds(ebase, ET)],
                              dst1.at[pl.ds(0, ET)], gsems[3])

    def zloop(i, carry):
        zbuf[i, :] = zero16
        den_loc[pl.ds(i * 16, 16)] = zero16
        return carry
    lax.fori_loop(0, NPAD // 16, zloop, 0)

    pltpu.sync_copy(zbuf, acc_sh.at[pl.ds(sid * NODES_PER_TILE, NODES_PER_TILE)])
    cp_dst.wait()

    def rloop(i, carry):
        v = dst1[pl.ds(i * 16, 16)]
        dst2d[i // 8, pl.ds((i % 8) * 16, 16)] = v
        return carry
    lax.fori_loop(0, ETP // 16, rloop, 0)
    cp_s.wait()
    cp_d.wait()
    cp_src.wait()
    plsc.subcore_barrier()

    bufs = [(zrows[b], wrows[b], gsems[b], ssems[b]) for b in range(NDEPTH)]
    for b, (zb, wb, gs, ss) in enumerate(bufs):
        pltpu.async_copy(z_hbm.at[src1.at[pl.ds(b * 128, 128)]], zb, gs)

    def body2(jj, carry):
        for b, (zb, wb, gs, ss) in enumerate(bufs):
            j = jj * NDEPTH + b
            pltpu.make_async_copy(z_hbm.at[src1.at[pl.ds(0, 128)]], zb, gs).wait()

            @pl.when(jj > 0)
            def _():
                pltpu.make_async_copy(wb, acc_sh.at[dst2d.at[j]], ss).wait()

            @plsc.parallel_loop(0, 8, 1, unroll=2)
            def chunk(k):
                off = j * 128 + k * 16
                si = src1[pl.ds(off, 16)]
                di = dst1[pl.ds(off, 16)]
                sv = plsc.load_gather(s_loc, [si])
                dv = plsc.load_gather(d_loc, [di])
                e = sv + dv
                e = jnp.where(e > 0, e, e * jnp.float32(0.01))
                ex = jnp.exp(e)
                plsc.addupdate_scatter(den_loc, [di], ex)
                base = k * 16
                for t in range(16):
                    wb[base + t, :] = zb[base + t, :] * ex[t]
            pltpu.async_copy(wb, acc_sh.at[dst2d.at[j]], ss, add=True)

            @pl.when(j + NDEPTH < ROWS)
            def _():
                pltpu.async_copy(
                    z_hbm.at[src1.at[pl.ds((j + NDEPTH) * 128, 128)]], zb, gs)
        return carry
    lax.fori_loop(0, ROWS // NDEPTH, body2, 0)
    pltpu.sync_copy(den_loc, denp.at[cid, sid])
    for b, (zb, wb, gs, ss) in enumerate(bufs):
        pltpu.make_async_copy(wb, acc_sh.at[dst2d.at[b]], ss).wait()
    plsc.subcore_barrier()

    nbase = sid * NODES_PER_TILE
    pltpu.sync_copy(acc_sh.at[pl.ds(nbase, NODES_PER_TILE)],
                    accp.at[cid, pl.ds(nbase, NODES_PER_TILE)])


_edge_call = functools.partial(
    pl.kernel,
    out_type=(jax.ShapeDtypeStruct((NC, NPAD, OUT_DIM), jnp.float32),
              jax.ShapeDtypeStruct((NC, NS, NPAD), jnp.float32)),
    mesh=plsc.VectorSubcoreMesh(core_axis_name="c", subcore_axis_name="s",
                                num_cores=NC, num_subcores=NS),
    scratch_types=[
        pltpu.VMEM((NPAD,), jnp.float32),
        pltpu.VMEM((NPAD,), jnp.float32),
        pltpu.VMEM((ETP,), jnp.int32),
        pltpu.VMEM((ETP,), jnp.int32),
        pltpu.VMEM((ROWS, 128), jnp.int32),
        pltpu.VMEM((NPAD,), jnp.float32),
        [pltpu.VMEM((128, OUT_DIM), jnp.float32) for _ in range(NDEPTH)],
        [pltpu.VMEM((128, OUT_DIM), jnp.float32) for _ in range(NDEPTH)],
        pltpu.VMEM((NODES_PER_TILE, OUT_DIM), jnp.float32),
        pltpu.VMEM_SHARED((NPAD, OUT_DIM), jnp.float32),
        [pltpu.SemaphoreType.DMA for _ in range(NDEPTH)],
        [pltpu.SemaphoreType.DMA for _ in range(NDEPTH)],
    ],
    compiler_params=pltpu.CompilerParams(needs_layout_passes=False,
                                         use_tc_tiling_on_sc=False),
)(_edge_body)


def _fin_body(accp, denp, out, a0b, a1b, denb, outb, sem0, sem1, sem2):
    cid = lax.axis_index("c")
    sid = lax.axis_index("s")
    wid = cid * NS + sid
    nbase = wid * FIN_PER_TILE

    cp0 = pltpu.async_copy(accp.at[0, pl.ds(nbase, FIN_PER_TILE)], a0b, sem0)
    cp1 = pltpu.async_copy(accp.at[1, pl.ds(nbase, FIN_PER_TILE)], a1b, sem1)
    cp2 = pltpu.async_copy(denp.at[:, :, pl.ds(nbase, FIN_PER_TILE)], denb, sem2)
    cp0.wait()
    cp1.wait()
    cp2.wait()

    eps = jnp.float32(1e-16)
    one = jnp.float32(1.0)

    def body(v, carry):
        dtot = denb[0, 0, pl.ds(v * 16, 16)]
        for c in range(NC):
            for t in range(NS):
                if c == 0 and t == 0:
                    continue
                dtot = dtot + denb[c, t, pl.ds(v * 16, 16)]
        dtot = dtot + eps
        base = v * 16
        for t in range(16):
            y = (a0b[base + t, :] + a1b[base + t, :]) / dtot[t]
            y = jnp.where(y > 0, y, jnp.exp(y) - one)
            outb[base + t, :] = y
        return carry
    lax.fori_loop(0, FIN_PER_TILE // 16, body, 0)

    @pl.when(nbase + FIN_PER_TILE <= N)
    def _():
        pltpu.sync_copy(outb, out.at[pl.ds(nbase, FIN_PER_TILE)])

    @pl.when(nbase + FIN_PER_TILE > N)
    def _():
        pltpu.sync_copy(outb.at[pl.ds(0, N - (NW - 1) * FIN_PER_TILE)],
                        out.at[pl.ds((NW - 1) * FIN_PER_TILE,
                                     N - (NW - 1) * FIN_PER_TILE)])


_fin_call = functools.partial(
    pl.kernel,
    out_type=jax.ShapeDtypeStruct((N, OUT_DIM), jnp.float32),
    mesh=plsc.VectorSubcoreMesh(core_axis_name="c", subcore_axis_name="s",
                                num_cores=NC, num_subcores=NS),
    scratch_types=[
        pltpu.VMEM((FIN_PER_TILE, OUT_DIM), jnp.float32),
        pltpu.VMEM((FIN_PER_TILE, OUT_DIM), jnp.float32),
        pltpu.VMEM((NC, NS, FIN_PER_TILE), jnp.float32),
        pltpu.VMEM((FIN_PER_TILE, OUT_DIM), jnp.float32),
        pltpu.SemaphoreType.DMA,
        pltpu.SemaphoreType.DMA,
        pltpu.SemaphoreType.DMA,
    ],
    compiler_params=pltpu.CompilerParams(needs_layout_passes=False,
                                         use_tc_tiling_on_sc=False),
)(_fin_body)


def kernel(h, edge_index, threshold, W, A):
    th2 = jnp.reshape(threshold.astype(jnp.float32), (1, 1))
    z, s, d = _zsd_call(th2, h, W, A)
    accp, denp = _edge_call(edge_index, z, s, d)
    return _fin_call(accp, denp)


# speedup vs baseline: 1.0486x; 1.0165x over previous
import functools

import jax
import jax.numpy as jnp
from jax import lax
from jax.experimental import pallas as pl
from jax.experimental.pallas import tpu as pltpu
from jax.experimental.pallas import tpu_sc as plsc

N = 10000
E = 320000
IN_DIM = 128
OUT_DIM = 16

NC = 2
NS = 16
NW = NC * NS
NPAD = 10240
ET = E // NW
ETP = 10240
ROWS = ETP // 128
NDEPTH = 4
NODES_PER_TILE = NPAD // NS
FIN_PER_TILE = NPAD // NW
BS = 2048


def _zsd_body(th_ref, h_ref, w_ref, a_ref, z_ref, s_ref, d_ref):
    th = th_ref[0, 0]
    w = w_ref[...]
    wm = w * (jnp.abs(w) > th).astype(w.dtype)
    z = lax.dot_general(h_ref[...], wm, (((1,), (1,)), ((), ())),
                        preferred_element_type=jnp.float32)
    a = a_ref[...]
    am = a * (jnp.abs(a) > th).astype(a.dtype)
    a1 = am[:, :OUT_DIM]
    a2 = am[:, OUT_DIM:]
    z_ref[...] = z
    s_ref[...] = jnp.sum(z * a1, axis=1)
    d_ref[...] = jnp.sum(z * a2, axis=1)


_zsd_call = pl.pallas_call(
    _zsd_body,
    grid=(NPAD // BS,),
    in_specs=[
        pl.BlockSpec((1, 1), lambda i: (0, 0)),
        pl.BlockSpec((BS, IN_DIM), lambda i: (i, 0)),
        pl.BlockSpec((OUT_DIM, IN_DIM), lambda i: (0, 0)),
        pl.BlockSpec((1, 2 * OUT_DIM), lambda i: (0, 0)),
    ],
    out_specs=[
        pl.BlockSpec((BS, OUT_DIM), lambda i: (i, 0)),
        pl.BlockSpec((BS,), lambda i: (i,)),
        pl.BlockSpec((BS,), lambda i: (i,)),
    ],
    out_shape=[
        jax.ShapeDtypeStruct((NPAD, OUT_DIM), jnp.float32),
        jax.ShapeDtypeStruct((NPAD,), jnp.float32),
        jax.ShapeDtypeStruct((NPAD,), jnp.float32),
    ],
)


def _edge_body(edge_hbm, z_hbm, s_hbm, d_hbm, accp, denp,
               s_loc, d_loc, src1, dst1, dst2d, den_loc,
               zrows, wrows, zbuf, acc_sh, gsems, ssems):
    cid = lax.axis_index("c")
    sid = lax.axis_index("s")
    wid = cid * NS + sid
    ebase = wid * ET
    zero16 = jnp.zeros((16,), jnp.float32)
    zero16i = jnp.zeros((16,), jnp.int32)

    padn = jnp.full((16,), N, jnp.int32)
    def ztail(i, carry):
        src1[pl.ds(ET + i * 16, 16)] = zero16i
        dst1[pl.ds(ET + i * 16, 16)] = padn
        s_loc[pl.ds(N + i * 16, 16)] = zero16
        d_loc[pl.ds(N + i * 16, 16)] = zero16
        return carry
    lax.fori_loop(0, (ETP - ET) // 16, ztail, 0)

    cp_s = pltpu.async_copy(s_hbm.at[pl.ds(0, N)], s_loc.at[pl.ds(0, N)], gsems[0])
    cp_d = pltpu.async_copy(d_hbm.at[pl.ds(0, N)], d_loc.at[pl.ds(0, N)], gsems[1])
    cp_src = pltpu.async_copy(edge_hbm.at[0, pl.ds(ebase, ET)],
                              src1.at[pl.ds(0, ET)], gsems[2])
    cp_dst = pltpu.async_copy(edge_hbm.at[1, pl.ds(ebase, ET)],
                              dst1.at[pl.ds(0, ET)], gsems[3])

    def zloop(i, carry):
        zbuf[i, :] = zero16
        den_loc[pl.ds(i * 16, 16)] = zero16
        return carry
    lax.fori_loop(0, NPAD // 16, zloop, 0)

    pltpu.sync_copy(zbuf, acc_sh.at[pl.ds(sid * NODES_PER_TILE, NODES_PER_TILE)])
    cp_dst.wait()

    def rloop(i, carry):
        v = dst1[pl.ds(i * 16, 16)]
        dst2d[i // 8, pl.ds((i % 8) * 16, 16)] = v
        return carry
    lax.fori_loop(0, ETP // 16, rloop, 0)
    cp_s.wait()
    cp_d.wait()
    cp_src.wait()
    plsc.subcore_barrier()

    bufs = [(zrows[b], wrows[b], gsems[b], ssems[b]) for b in range(NDEPTH)]
    for b, (zb, wb, gs, ss) in enumerate(bufs):
        pltpu.async_copy(z_hbm.at[src1.at[pl.ds(b * 128, 128)]], zb, gs)

    def body2(jj, carry):
        for b, (zb, wb, gs, ss) in enumerate(bufs):
            j = jj * NDEPTH + b
            pltpu.make_async_copy(z_hbm.at[src1.at[pl.ds(0, 128)]], zb, gs).wait()

            @pl.when(jj > 0)
            def _():
                pltpu.make_async_copy(wb, acc_sh.at[dst2d.at[j]], ss).wait()

            @plsc.parallel_loop(0, 8, 1, unroll=4)
            def chunk(k):
                off = j * 128 + k * 16
                si = src1[pl.ds(off, 16)]
                di = dst1[pl.ds(off, 16)]
                sv = plsc.load_gather(s_loc, [si])
                dv = plsc.load_gather(d_loc, [di])
                e = sv + dv
                e = jnp.where(e > 0, e, e * jnp.float32(0.01))
                ex = jnp.exp(e)
                plsc.addupdate_scatter(den_loc, [di], ex)
                base = k * 16
                for t in range(16):
                    wb[base + t, :] = zb[base + t, :] * ex[t]
            pltpu.async_copy(wb, acc_sh.at[dst2d.at[j]], ss, add=True)

            @pl.when(j + NDEPTH < ROWS)
            def _():
                pltpu.async_copy(
                    z_hbm.at[src1.at[pl.ds((j + NDEPTH) * 128, 128)]], zb, gs)
        return carry
    lax.fori_loop(0, ROWS // NDEPTH, body2, 0)
    pltpu.sync_copy(den_loc, denp.at[cid, sid])
    for b, (zb, wb, gs, ss) in enumerate(bufs):
        pltpu.make_async_copy(wb, acc_sh.at[dst2d.at[b]], ss).wait()
    plsc.subcore_barrier()

    nbase = sid * NODES_PER_TILE
    pltpu.sync_copy(acc_sh.at[pl.ds(nbase, NODES_PER_TILE)],
                    accp.at[cid, pl.ds(nbase, NODES_PER_TILE)])


_edge_call = functools.partial(
    pl.kernel,
    out_type=(jax.ShapeDtypeStruct((NC, NPAD, OUT_DIM), jnp.float32),
              jax.ShapeDtypeStruct((NC, NS, NPAD), jnp.float32)),
    mesh=plsc.VectorSubcoreMesh(core_axis_name="c", subcore_axis_name="s",
                                num_cores=NC, num_subcores=NS),
    scratch_types=[
        pltpu.VMEM((NPAD,), jnp.float32),
        pltpu.VMEM((NPAD,), jnp.float32),
        pltpu.VMEM((ETP,), jnp.int32),
        pltpu.VMEM((ETP,), jnp.int32),
        pltpu.VMEM((ROWS, 128), jnp.int32),
        pltpu.VMEM((NPAD,), jnp.float32),
        [pltpu.VMEM((128, OUT_DIM), jnp.float32) for _ in range(NDEPTH)],
        [pltpu.VMEM((128, OUT_DIM), jnp.float32) for _ in range(NDEPTH)],
        pltpu.VMEM((NODES_PER_TILE, OUT_DIM), jnp.float32),
        pltpu.VMEM_SHARED((NPAD, OUT_DIM), jnp.float32),
        [pltpu.SemaphoreType.DMA for _ in range(NDEPTH)],
        [pltpu.SemaphoreType.DMA for _ in range(NDEPTH)],
    ],
    compiler_params=pltpu.CompilerParams(needs_layout_passes=False,
                                         use_tc_tiling_on_sc=False),
)(_edge_body)


def _fin_body(accp, denp, out, a0b, a1b, denb, outb, sem0, sem1, sem2):
    cid = lax.axis_index("c")
    sid = lax.axis_index("s")
    wid = cid * NS + sid
    nbase = wid * FIN_PER_TILE

    cp0 = pltpu.async_copy(accp.at[0, pl.ds(nbase, FIN_PER_TILE)], a0b, sem0)
    cp1 = pltpu.async_copy(accp.at[1, pl.ds(nbase, FIN_PER_TILE)], a1b, sem1)
    cp2 = pltpu.async_copy(denp.at[:, :, pl.ds(nbase, FIN_PER_TILE)], denb, sem2)
    cp0.wait()
    cp1.wait()
    cp2.wait()

    eps = jnp.float32(1e-16)
    one = jnp.float32(1.0)

    def body(v, carry):
        dtot = denb[0, 0, pl.ds(v * 16, 16)]
        for c in range(NC):
            for t in range(NS):
                if c == 0 and t == 0:
                    continue
                dtot = dtot + denb[c, t, pl.ds(v * 16, 16)]
        dtot = dtot + eps
        base = v * 16
        for t in range(16):
            y = (a0b[base + t, :] + a1b[base + t, :]) / dtot[t]
            y = jnp.where(y > 0, y, jnp.exp(y) - one)
            outb[base + t, :] = y
        return carry
    lax.fori_loop(0, FIN_PER_TILE // 16, body, 0)

    @pl.when(nbase + FIN_PER_TILE <= N)
    def _():
        pltpu.sync_copy(outb, out.at[pl.ds(nbase, FIN_PER_TILE)])

    @pl.when(nbase + FIN_PER_TILE > N)
    def _():
        pltpu.sync_copy(outb.at[pl.ds(0, N - (NW - 1) * FIN_PER_TILE)],
                        out.at[pl.ds((NW - 1) * FIN_PER_TILE,
                                     N - (NW - 1) * FIN_PER_TILE)])


_fin_call = functools.partial(
    pl.kernel,
    out_type=jax.ShapeDtypeStruct((N, OUT_DIM), jnp.float32),
    mesh=plsc.VectorSubcoreMesh(core_axis_name="c", subcore_axis_name="s",
                                num_cores=NC, num_subcores=NS),
    scratch_types=[
        pltpu.VMEM((FIN_PER_TILE, OUT_DIM), jnp.float32),
        pltpu.VMEM((FIN_PER_TILE, OUT_DIM), jnp.float32),
        pltpu.VMEM((NC, NS, FIN_PER_TILE), jnp.float32),
        pltpu.VMEM((FIN_PER_TILE, OUT_DIM), jnp.float32),
        pltpu.SemaphoreType.DMA,
        pltpu.SemaphoreType.DMA,
        pltpu.SemaphoreType.DMA,
    ],
    compiler_params=pltpu.CompilerParams(needs_layout_passes=False,
                                         use_tc_tiling_on_sc=False),
)(_fin_body)


def kernel(h, edge_index, threshold, W, A):
    th2 = jnp.reshape(threshold.astype(jnp.float32), (1, 1))
    z, s, d = _zsd_call(th2, h, W, A)
    accp, denp = _edge_call(edge_index, z, s, d)
    return _fin_call(accp, denp)
